# Initial kernel scaffold; baseline (speedup 1.0000x reference)
import functools

import jax
import jax.numpy as jnp
from jax import lax
from jax.experimental import pallas as pl
from jax.experimental.pallas import tpu as pltpu
from jax.experimental.pallas import tpu_sc as plsc

_NC = 2
_NS = 16
_NW = _NC * _NS
_LANES = 16


@functools.lru_cache(maxsize=None)
def _build(n_rows, d, s, ch):
    n_per_w = n_rows // _NW
    n_chunks = n_per_w // ch
    mesh = plsc.VectorSubcoreMesh(
        core_axis_name="c", subcore_axis_name="s",
        num_cores=_NC, num_subcores=_NS,
    )

    @functools.partial(
        pl.kernel,
        out_type=jax.ShapeDtypeStruct((n_rows, d), jnp.float32),
        mesh=mesh,
        scratch_types=[
            pltpu.VMEM((ch,), jnp.int32),
            pltpu.VMEM((ch, d), jnp.float32),
            pltpu.VMEM((s, d), jnp.float32),
            pltpu.SemaphoreType.DMA,
        ],
    )
    def emb_kernel(tok_hbm, table_hbm, pos_hbm, out_hbm, idx_v, rows_v, pos_v, sem):
        wid = lax.axis_index("s") * _NC + lax.axis_index("c")
        base = wid * n_per_w
        pltpu.sync_copy(pos_hbm, pos_v)

        def chunk_body(g, carry):
            cbase = base + g * ch
            pltpu.sync_copy(tok_hbm.at[pl.ds(cbase, ch)], idx_v)
            pltpu.async_copy(table_hbm.at[idx_v], rows_v, sem).wait()

            def row_body(r, c2):
                for c in range(d // _LANES):
                    sl = pl.ds(c * _LANES, _LANES)
                    rows_v[r, sl] = rows_v[r, sl] + pos_v[r, sl]
                return c2

            lax.fori_loop(0, ch, row_body, 0)
            pltpu.sync_copy(rows_v, out_hbm.at[pl.ds(cbase, ch)])
            return carry

        lax.fori_loop(0, n_chunks, chunk_body, 0)

    return emb_kernel


def kernel(tokens, token_embedding, position_embedding):
    b, s = tokens.shape
    _, d = token_embedding.shape
    flat = tokens.reshape(-1).astype(jnp.int32)
    fn = _build(b * s, d, s, s)
    out = fn(flat, token_embedding, position_embedding[:s])
    return out.reshape(b, s, d)


# speedup vs baseline: 3.8121x; 3.8121x over previous
import functools

import jax
import jax.numpy as jnp
from jax import lax
from jax.experimental import pallas as pl
from jax.experimental.pallas import tpu as pltpu
from jax.experimental.pallas import tpu_sc as plsc

_NC = 2
_NS = 16
_NW = _NC * _NS
_LANES = 16


@functools.lru_cache(maxsize=None)
def _build(n_rows, d, s, ch):
    n_per_w = n_rows // _NW
    n_chunks = n_per_w // ch
    mesh = plsc.VectorSubcoreMesh(
        core_axis_name="c", subcore_axis_name="s",
        num_cores=_NC, num_subcores=_NS,
    )

    @functools.partial(
        pl.kernel,
        out_type=jax.ShapeDtypeStruct((n_rows, d), jnp.float32),
        mesh=mesh,
        scratch_types=[
            pltpu.VMEM((ch,), jnp.int32),
            pltpu.VMEM((ch, 128), jnp.float32),
            pltpu.VMEM((ch, d), jnp.float32),
            pltpu.VMEM((s, d), jnp.float32),
            pltpu.SemaphoreType.DMA,
        ],
    )
    def emb_kernel(tok_hbm, table_hbm, pos_hbm, out_hbm, idx_v, rows_v, out_v, pos_v, sem):
        wid = lax.axis_index("s") * _NC + lax.axis_index("c")
        base = wid * n_per_w
        pltpu.sync_copy(pos_hbm, pos_v)

        def chunk_body(g, carry):
            cbase = base + g * ch
            pltpu.sync_copy(tok_hbm.at[pl.ds(cbase, ch)], idx_v)
            pltpu.async_copy(table_hbm.at[idx_v], rows_v, sem).wait()

            def row_body(r, c2):
                for c in range(d // _LANES):
                    sl = pl.ds(c * _LANES, _LANES)
                    out_v[r, sl] = rows_v[r, sl] + pos_v[r, sl]
                return c2

            lax.fori_loop(0, ch, row_body, 0)
            pltpu.sync_copy(out_v, out_hbm.at[pl.ds(cbase, ch)])
            return carry

        lax.fori_loop(0, n_chunks, chunk_body, 0)

    return emb_kernel


def kernel(tokens, token_embedding, position_embedding):
    b, s = tokens.shape
    _, d = token_embedding.shape
    flat = tokens.reshape(-1).astype(jnp.int32)
    table128 = jnp.pad(token_embedding, ((0, 0), (0, 128 - d)))
    fn = _build(b * s, d, s, s)
    out = fn(flat, table128, position_embedding[:s])
    return out.reshape(b, s, d)
